# baseline (speedup 1.0000x reference)
import functools

import jax
import jax.numpy as jnp
from jax import lax
from jax.experimental import pallas as pl
from jax.experimental.pallas import tpu as pltpu
from jax.experimental.pallas import tpu_sc as plsc

N = 10000
E = 320000
FT = 128
H = 64
STEPS = 6
S2S_STEPS = 6

_NC = 2
_NS = 16
_NW = _NC * _NS
_CH = 128
_RING = 6
_C0 = 30
_C1 = 132
_CMAX = max(_C0, _C1)
_TOTCH = _NS * (_C0 + _C1)
_EPAD = _TOTCH * _CH
_NACC = N + 16
_ZROWS = _NACC // _NS
_OROWS = N // _NS


def _sc_scatter_body(msg_hbm, src_hbm, dst_hbm, out_hbm,
                     srcv, dstv, rows, acc, *sems):
    c = lax.axis_index("c")
    s = lax.axis_index("s")
    gsems = sems[:_RING]
    ssems = sems[_RING:]

    nch = jnp.where(c == 0, _C0, _C1)
    start = jnp.where(c == 0, s * _C0, _NS * _C0 + s * _C1)
    pltpu.sync_copy(src_hbm.at[pl.ds(start, _CMAX)], srcv)
    pltpu.sync_copy(dst_hbm.at[pl.ds(start, _CMAX)], dstv)

    def _zrow(i, carry):
        for k in range(4):
            rows[0, i, pl.ds(k * 16, 16)] = jnp.zeros((16,), jnp.float32)
        return carry
    lax.fori_loop(0, _CH, _zrow, 0)
    zb = s * _ZROWS
    nfull = _ZROWS // _CH
    for q in range(nfull):
        pltpu.sync_copy(rows.at[0], acc.at[pl.ds(zb + q * _CH, _CH)])
    rem = _ZROWS - nfull * _CH
    if rem:
        pltpu.sync_copy(rows.at[0, pl.ds(0, rem)],
                        acc.at[pl.ds(zb + nfull * _CH, rem)])
    plsc.subcore_barrier()

    def _wait_scatter(b):
        pltpu.make_async_copy(
            rows.at[b], acc.at[dstv.at[0]], ssems[b]).wait()

    def _group(g, carry):
        gds = []
        for b in range(_RING):
            j = g * _RING + b
            lax.cond(g > 0, lambda b=b: _wait_scatter(b), lambda: None)
            gds.append(pltpu.async_copy(
                msg_hbm.at[srcv.at[j]], rows.at[b], gsems[b]))
        for b in range(_RING):
            j = g * _RING + b
            gds[b].wait()
            pltpu.async_copy(rows.at[b], acc.at[dstv.at[j]], ssems[b],
                             add=True)
        return carry
    lax.fori_loop(0, nch // _RING, _group, 0)
    for b in range(_RING):
        _wait_scatter(b)

    plsc.subcore_barrier()
    ob = s * _OROWS
    pltpu.sync_copy(acc.at[pl.ds(ob, _OROWS)], out_hbm.at[c, pl.ds(ob, _OROWS)])


@functools.cache
def _get_sc_scatter():
    return functools.partial(
        pl.kernel,
        out_type=jax.ShapeDtypeStruct((_NC, N, H), jnp.float32),
        mesh=plsc.VectorSubcoreMesh(core_axis_name="c", subcore_axis_name="s",
                                    num_cores=_NC, num_subcores=_NS),
        scratch_types=[
            pltpu.VMEM((_CMAX, _CH), jnp.int32),
            pltpu.VMEM((_CMAX, _CH), jnp.int32),
            pltpu.VMEM((_RING, _CH, H), jnp.float32),
            pltpu.VMEM_SHARED((_NACC, H), jnp.float32),
        ] + [pltpu.SemaphoreType.DMA] * (2 * _RING),
        compiler_params=pltpu.CompilerParams(use_tc_tiling_on_sc=False),
    )(_sc_scatter_body)


def _encode_body(x_ref, w0_ref, b0_ref, wmsg_ref, out_ref, msg_ref):
    o = jnp.maximum(x_ref[...] @ w0_ref[...] + b0_ref[...], 0.0)
    out_ref[...] = o
    msg_ref[...] = o @ wmsg_ref[...]


_encode = pl.pallas_call(
    _encode_body,
    out_shape=[jax.ShapeDtypeStruct((N, H), jnp.float32),
               jax.ShapeDtypeStruct((N, H), jnp.float32)],
)


def _gru_body(p_ref, h_ref, wi_ref, wh_ref, bi_ref, bh_ref, wmsg_ref,
              h_out_ref, msg_ref, *, last):
    m = jnp.maximum(p_ref[0] + p_ref[1], 0.0)
    h = h_ref[...]
    gx = m @ wi_ref[...] + bi_ref[...]
    gh = h @ wh_ref[...] + bh_ref[...]
    r = jax.nn.sigmoid(gx[:, :H] + gh[:, :H])
    z = jax.nn.sigmoid(gx[:, H:2 * H] + gh[:, H:2 * H])
    n = jnp.tanh(gx[:, 2 * H:] + r * gh[:, 2 * H:])
    hn = (1.0 - z) * n + z * h
    h_out_ref[...] = hn
    if last:
        msg_ref[...] = jnp.zeros_like(hn)
    else:
        msg_ref[...] = hn @ wmsg_ref[...]


_gru = pl.pallas_call(
    functools.partial(_gru_body, last=False),
    out_shape=[jax.ShapeDtypeStruct((N, H), jnp.float32),
               jax.ShapeDtypeStruct((N, H), jnp.float32)],
)

_gru_last = pl.pallas_call(
    functools.partial(_gru_body, last=True),
    out_shape=[jax.ShapeDtypeStruct((N, H), jnp.float32),
               jax.ShapeDtypeStruct((N, H), jnp.float32)],
)


def _s2s_body(out_ref, wi0, wh0, b0r, wi1, wh1, b1r, wi2, wh2, b2r,
              wr1, br1, wr2, br2, y_ref):
    out = out_ref[...]
    wi = (wi0[...], wi1[...], wi2[...])
    wh = (wh0[...], wh1[...], wh2[...])
    bb = (b0r[...], b1r[...], b2r[...])
    hs = [jnp.zeros((1, H), jnp.float32) for _ in range(3)]
    cs = [jnp.zeros((1, H), jnp.float32) for _ in range(3)]
    q_star = jnp.zeros((1, 2 * H), jnp.float32)
    for _ in range(S2S_STEPS):
        inp_t = q_star
        for l in range(3):
            gates = inp_t @ wi[l] + hs[l] @ wh[l] + bb[l]
            i = jax.nn.sigmoid(gates[:, :H])
            f = jax.nn.sigmoid(gates[:, H:2 * H])
            g = jnp.tanh(gates[:, 2 * H:3 * H])
            o = jax.nn.sigmoid(gates[:, 3 * H:])
            c = f * cs[l] + i * g
            hh = o * jnp.tanh(c)
            hs[l] = hh
            cs[l] = c
            inp_t = hh
        q = hs[2]
        e = out @ q.T
        emax = jnp.max(e, axis=0, keepdims=True)
        ex = jnp.exp(e - emax)
        alpha = ex / jnp.sum(ex, axis=0, keepdims=True)
        rvec = jnp.sum(alpha * out, axis=0, keepdims=True)
        q_star = jnp.concatenate([q, rvec], axis=1)
    y = jnp.maximum(q_star @ wr1[...] + br1[...], 0.0)
    y_ref[...] = y @ wr2[...] + br2[...]


_s2s = pl.pallas_call(
    _s2s_body,
    out_shape=jax.ShapeDtypeStruct((1, H), jnp.float32),
)


def kernel(x, edge_index, W0, b0, Wmsg, gru_Wi, gru_Wh, gru_bi, gru_bh,
           lstm_Wi0, lstm_Wh0, lstm_b0, lstm_Wi1, lstm_Wh1, lstm_b1,
           lstm_Wi2, lstm_Wh2, lstm_b2, Wr1, br1, Wr2, br2):
    src = edge_index[0]
    dst = edge_index[1]
    npad = _EPAD - E
    srcr = jnp.concatenate(
        [src, jnp.zeros((npad,), jnp.int32)]).reshape(_TOTCH, _CH)
    dstr = jnp.concatenate(
        [dst, jnp.full((npad,), N, jnp.int32)]).reshape(_TOTCH, _CH)

    out0, msg = _encode(x, W0, b0.reshape(1, H), Wmsg)
    h = out0
    bi = gru_bi.reshape(1, 3 * H)
    bh = gru_bh.reshape(1, 3 * H)
    for step in range(STEPS):
        partial = _get_sc_scatter()(msg, srcr, dstr)
        call = _gru_last if step == STEPS - 1 else _gru
        h, msg = call(partial, h, gru_Wi, gru_Wh, bi, bh, Wmsg)
    y = _s2s(h, lstm_Wi0, lstm_Wh0, lstm_b0.reshape(1, 4 * H),
             lstm_Wi1, lstm_Wh1, lstm_b1.reshape(1, 4 * H),
             lstm_Wi2, lstm_Wh2, lstm_b2.reshape(1, 4 * H),
             Wr1, br1.reshape(1, H), Wr2, br2.reshape(1, H))
    return y


# speedup vs baseline: 1.1708x; 1.1708x over previous
import functools

import jax
import jax.numpy as jnp
from jax import lax
from jax.experimental import pallas as pl
from jax.experimental.pallas import tpu as pltpu
from jax.experimental.pallas import tpu_sc as plsc

N = 10000
E = 320000
FT = 128
H = 64
STEPS = 6
S2S_STEPS = 6

_NC = 2
_NS = 16
_NW = _NC * _NS
_CH = 128
_RING = 6
_C0 = 132
_C1 = 30
_CMAX = max(_C0, _C1)
_TOTCH = _NS * (_C0 + _C1)
_TOTPAD = _TOTCH + _CMAX - min(_C0, _C1)
_EPAD = _TOTPAD * _CH
_NACC = N + 16
_ZROWS = _NACC // _NS
_OROWS = N // _NS


def _sc_scatter_body(msg_hbm, src_hbm, dst_hbm, out_hbm,
                     srcv, dstv, rows, acc, *sems):
    c = lax.axis_index("c")
    s = lax.axis_index("s")
    gsems = sems[:_RING]
    ssems = sems[_RING:]

    nch = jnp.where(c == 0, _C0, _C1)
    start = jnp.where(c == 0, s * _C0, _NS * _C0 + s * _C1)
    pltpu.sync_copy(src_hbm.at[pl.ds(start, _CMAX)], srcv)
    pltpu.sync_copy(dst_hbm.at[pl.ds(start, _CMAX)], dstv)

    def _zrow(i, carry):
        for k in range(4):
            rows[0, i, pl.ds(k * 16, 16)] = jnp.zeros((16,), jnp.float32)
        return carry
    lax.fori_loop(0, _CH, _zrow, 0)
    zb = s * _ZROWS
    nfull = _ZROWS // _CH
    for q in range(nfull):
        pltpu.sync_copy(rows.at[0], acc.at[pl.ds(zb + q * _CH, _CH)])
    rem = _ZROWS - nfull * _CH
    if rem:
        pltpu.sync_copy(rows.at[0, pl.ds(0, rem)],
                        acc.at[pl.ds(zb + nfull * _CH, rem)])
    plsc.subcore_barrier()

    def _wait_scatter(b):
        pltpu.make_async_copy(
            rows.at[b], acc.at[dstv.at[0]], ssems[b]).wait()

    def _group(g, carry):
        gds = []
        for b in range(_RING):
            j = g * _RING + b
            lax.cond(g > 0, lambda b=b: _wait_scatter(b), lambda: None)
            gds.append(pltpu.async_copy(
                msg_hbm.at[srcv.at[j]], rows.at[b], gsems[b]))
        for b in range(_RING):
            j = g * _RING + b
            gds[b].wait()
            pltpu.async_copy(rows.at[b], acc.at[dstv.at[j]], ssems[b],
                             add=True)
        return carry
    lax.fori_loop(0, nch // _RING, _group, 0)
    for b in range(_RING):
        _wait_scatter(b)

    plsc.subcore_barrier()
    ob = s * _OROWS
    pltpu.sync_copy(acc.at[pl.ds(ob, _OROWS)], out_hbm.at[c, pl.ds(ob, _OROWS)])


@functools.cache
def _get_sc_scatter():
    return functools.partial(
        pl.kernel,
        out_type=jax.ShapeDtypeStruct((_NC, N, H), jnp.float32),
        mesh=plsc.VectorSubcoreMesh(core_axis_name="c", subcore_axis_name="s",
                                    num_cores=_NC, num_subcores=_NS),
        scratch_types=[
            pltpu.VMEM((_CMAX, _CH), jnp.int32),
            pltpu.VMEM((_CMAX, _CH), jnp.int32),
            pltpu.VMEM((_RING, _CH, H), jnp.float32),
            pltpu.VMEM_SHARED((_NACC, H), jnp.float32),
        ] + [pltpu.SemaphoreType.DMA] * (2 * _RING),
        compiler_params=pltpu.CompilerParams(use_tc_tiling_on_sc=False),
    )(_sc_scatter_body)


def _encode_body(x_ref, w0_ref, b0_ref, wmsg_ref, out_ref, msg_ref):
    o = jnp.maximum(x_ref[...] @ w0_ref[...] + b0_ref[...], 0.0)
    out_ref[...] = o
    msg_ref[...] = o @ wmsg_ref[...]


_encode = pl.pallas_call(
    _encode_body,
    out_shape=[jax.ShapeDtypeStruct((N, H), jnp.float32),
               jax.ShapeDtypeStruct((N, H), jnp.float32)],
)


def _gru_body(p_ref, h_ref, wi_ref, wh_ref, bi_ref, bh_ref, wmsg_ref,
              h_out_ref, msg_ref, *, last):
    m = jnp.maximum(p_ref[0] + p_ref[1], 0.0)
    h = h_ref[...]
    gx = m @ wi_ref[...] + bi_ref[...]
    gh = h @ wh_ref[...] + bh_ref[...]
    r = jax.nn.sigmoid(gx[:, :H] + gh[:, :H])
    z = jax.nn.sigmoid(gx[:, H:2 * H] + gh[:, H:2 * H])
    n = jnp.tanh(gx[:, 2 * H:] + r * gh[:, 2 * H:])
    hn = (1.0 - z) * n + z * h
    h_out_ref[...] = hn
    if last:
        msg_ref[...] = jnp.zeros_like(hn)
    else:
        msg_ref[...] = hn @ wmsg_ref[...]


_gru = pl.pallas_call(
    functools.partial(_gru_body, last=False),
    out_shape=[jax.ShapeDtypeStruct((N, H), jnp.float32),
               jax.ShapeDtypeStruct((N, H), jnp.float32)],
)

_gru_last = pl.pallas_call(
    functools.partial(_gru_body, last=True),
    out_shape=[jax.ShapeDtypeStruct((N, H), jnp.float32),
               jax.ShapeDtypeStruct((N, H), jnp.float32)],
)


def _s2s_body(out_ref, wi0, wh0, b0r, wi1, wh1, b1r, wi2, wh2, b2r,
              wr1, br1, wr2, br2, y_ref):
    out = out_ref[...]
    wi = (wi0[...], wi1[...], wi2[...])
    wh = (wh0[...], wh1[...], wh2[...])
    bb = (b0r[...], b1r[...], b2r[...])
    hs = [jnp.zeros((1, H), jnp.float32) for _ in range(3)]
    cs = [jnp.zeros((1, H), jnp.float32) for _ in range(3)]
    q_star = jnp.zeros((1, 2 * H), jnp.float32)
    for _ in range(S2S_STEPS):
        inp_t = q_star
        for l in range(3):
            gates = inp_t @ wi[l] + hs[l] @ wh[l] + bb[l]
            i = jax.nn.sigmoid(gates[:, :H])
            f = jax.nn.sigmoid(gates[:, H:2 * H])
            g = jnp.tanh(gates[:, 2 * H:3 * H])
            o = jax.nn.sigmoid(gates[:, 3 * H:])
            c = f * cs[l] + i * g
            hh = o * jnp.tanh(c)
            hs[l] = hh
            cs[l] = c
            inp_t = hh
        q = hs[2]
        e = out @ q.T
        emax = jnp.max(e, axis=0, keepdims=True)
        ex = jnp.exp(e - emax)
        alpha = ex / jnp.sum(ex, axis=0, keepdims=True)
        rvec = jnp.sum(alpha * out, axis=0, keepdims=True)
        q_star = jnp.concatenate([q, rvec], axis=1)
    y = jnp.maximum(q_star @ wr1[...] + br1[...], 0.0)
    y_ref[...] = y @ wr2[...] + br2[...]


_s2s = pl.pallas_call(
    _s2s_body,
    out_shape=jax.ShapeDtypeStruct((1, H), jnp.float32),
)


def kernel(x, edge_index, W0, b0, Wmsg, gru_Wi, gru_Wh, gru_bi, gru_bh,
           lstm_Wi0, lstm_Wh0, lstm_b0, lstm_Wi1, lstm_Wh1, lstm_b1,
           lstm_Wi2, lstm_Wh2, lstm_b2, Wr1, br1, Wr2, br2):
    src = edge_index[0]
    dst = edge_index[1]
    npad = _EPAD - E
    srcr = jnp.concatenate(
        [src, jnp.zeros((npad,), jnp.int32)]).reshape(_TOTPAD, _CH)
    dstr = jnp.concatenate(
        [dst, jnp.full((npad,), N, jnp.int32)]).reshape(_TOTPAD, _CH)

    out0, msg = _encode(x, W0, b0.reshape(1, H), Wmsg)
    h = out0
    bi = gru_bi.reshape(1, 3 * H)
    bh = gru_bh.reshape(1, 3 * H)
    for step in range(STEPS):
        partial = _get_sc_scatter()(msg, srcr, dstr)
        call = _gru_last if step == STEPS - 1 else _gru
        h, msg = call(partial, h, gru_Wi, gru_Wh, bi, bh, Wmsg)
    y = _s2s(h, lstm_Wi0, lstm_Wh0, lstm_b0.reshape(1, 4 * H),
             lstm_Wi1, lstm_Wh1, lstm_b1.reshape(1, 4 * H),
             lstm_Wi2, lstm_Wh2, lstm_b2.reshape(1, 4 * H),
             Wr1, br1.reshape(1, H), Wr2, br2.reshape(1, H))
    return y


# speedup vs baseline: 1.3894x; 1.1868x over previous
import functools

import jax
import jax.numpy as jnp
from jax import lax
from jax.experimental import pallas as pl
from jax.experimental.pallas import tpu as pltpu
from jax.experimental.pallas import tpu_sc as plsc

N = 10000
E = 320000
FT = 128
H = 64
STEPS = 6
S2S_STEPS = 6

_NC = 2
_NS = 16
_NW = _NC * _NS
_CH = 128
_RING = 6
_NCHUNK = 80
_TOTCH = _NW * _NCHUNK
_EPAD = _TOTCH * _CH
_NACC = N + 16
_ZROWS = _NACC // _NS
_OROWS = N // _NS


def _sc_scatter_body(msg_hbm, src_hbm, dst_hbm, out_hbm,
                     srcv, dstv, rows, acc, *sems):
    c = lax.axis_index("c")
    s = lax.axis_index("s")
    gsems = sems[:_RING]
    ssems = sems[_RING:]

    start = (s * _NC + c) * _NCHUNK
    d1 = pltpu.async_copy(src_hbm.at[pl.ds(start, _NCHUNK)], srcv, gsems[0])
    d2 = pltpu.async_copy(dst_hbm.at[pl.ds(start, _NCHUNK)], dstv, gsems[1])

    def _zrow(i, carry):
        for k in range(4):
            rows[0, i, pl.ds(k * 16, 16)] = jnp.zeros((16,), jnp.float32)
        return carry
    lax.fori_loop(0, _CH, _zrow, 0)
    zb = s * _ZROWS
    nfull = _ZROWS // _CH
    for q in range(nfull):
        pltpu.sync_copy(rows.at[0], acc.at[pl.ds(zb + q * _CH, _CH)])
    rem = _ZROWS - nfull * _CH
    if rem:
        pltpu.sync_copy(rows.at[0, pl.ds(0, rem)],
                        acc.at[pl.ds(zb + nfull * _CH, rem)])
    d1.wait()
    d2.wait()
    plsc.subcore_barrier()

    def _wait_scatter(b):
        pltpu.make_async_copy(
            rows.at[b], acc.at[dstv.at[0]], ssems[b]).wait()

    def _group(g, carry):
        gds = []
        for b in range(_RING):
            j = g * _RING + b
            lax.cond(g > 0, lambda b=b: _wait_scatter(b), lambda: None)
            gds.append(pltpu.async_copy(
                msg_hbm.at[srcv.at[j]], rows.at[b], gsems[b]))
        for b in range(_RING):
            j = g * _RING + b
            gds[b].wait()
            pltpu.async_copy(rows.at[b], acc.at[dstv.at[j]], ssems[b],
                             add=True)
        return carry
    lax.fori_loop(0, _NCHUNK // _RING, _group, 0)
    for b in range(_RING):
        _wait_scatter(b)

    plsc.subcore_barrier()
    ob = s * _OROWS
    pltpu.sync_copy(acc.at[pl.ds(ob, _OROWS)], out_hbm.at[c, pl.ds(ob, _OROWS)])


@functools.cache
def _get_sc_scatter():
    return functools.partial(
        pl.kernel,
        out_type=jax.ShapeDtypeStruct((_NC, N, H), jnp.float32),
        mesh=plsc.VectorSubcoreMesh(core_axis_name="c", subcore_axis_name="s",
                                    num_cores=_NC, num_subcores=_NS),
        scratch_types=[
            pltpu.VMEM((_NCHUNK, _CH), jnp.int32),
            pltpu.VMEM((_NCHUNK, _CH), jnp.int32),
            pltpu.VMEM((_RING, _CH, H), jnp.float32),
            pltpu.VMEM_SHARED((_NACC, H), jnp.float32),
        ] + [pltpu.SemaphoreType.DMA] * (2 * _RING),
        compiler_params=pltpu.CompilerParams(use_tc_tiling_on_sc=False),
    )(_sc_scatter_body)


def _encode_body(x_ref, w0_ref, b0_ref, wmsg_ref, out_ref, msg_ref):
    o = jnp.maximum(x_ref[...] @ w0_ref[...] + b0_ref[...], 0.0)
    out_ref[...] = o
    msg_ref[...] = o @ wmsg_ref[...]


_encode = pl.pallas_call(
    _encode_body,
    out_shape=[jax.ShapeDtypeStruct((N, H), jnp.float32),
               jax.ShapeDtypeStruct((N, H), jnp.float32)],
)


def _gru_body(p_ref, h_ref, wi_ref, wh_ref, bi_ref, bh_ref, wmsg_ref,
              h_out_ref, msg_ref, *, last):
    m = jnp.maximum(p_ref[0] + p_ref[1], 0.0)
    h = h_ref[...]
    gx = m @ wi_ref[...] + bi_ref[...]
    gh = h @ wh_ref[...] + bh_ref[...]
    r = jax.nn.sigmoid(gx[:, :H] + gh[:, :H])
    z = jax.nn.sigmoid(gx[:, H:2 * H] + gh[:, H:2 * H])
    n = jnp.tanh(gx[:, 2 * H:] + r * gh[:, 2 * H:])
    hn = (1.0 - z) * n + z * h
    h_out_ref[...] = hn
    if last:
        msg_ref[...] = jnp.zeros_like(hn)
    else:
        msg_ref[...] = hn @ wmsg_ref[...]


_gru = pl.pallas_call(
    functools.partial(_gru_body, last=False),
    out_shape=[jax.ShapeDtypeStruct((N, H), jnp.float32),
               jax.ShapeDtypeStruct((N, H), jnp.float32)],
)

_gru_last = pl.pallas_call(
    functools.partial(_gru_body, last=True),
    out_shape=[jax.ShapeDtypeStruct((N, H), jnp.float32),
               jax.ShapeDtypeStruct((N, H), jnp.float32)],
)


def _s2s_body(out_ref, wi0, wh0, b0r, wi1, wh1, b1r, wi2, wh2, b2r,
              wr1, br1, wr2, br2, y_ref):
    out = out_ref[...]
    wi = (wi0[...], wi1[...], wi2[...])
    wh = (wh0[...], wh1[...], wh2[...])
    bb = (b0r[...], b1r[...], b2r[...])
    hs = [jnp.zeros((1, H), jnp.float32) for _ in range(3)]
    cs = [jnp.zeros((1, H), jnp.float32) for _ in range(3)]
    q_star = jnp.zeros((1, 2 * H), jnp.float32)
    for _ in range(S2S_STEPS):
        inp_t = q_star
        for l in range(3):
            gates = inp_t @ wi[l] + hs[l] @ wh[l] + bb[l]
            i = jax.nn.sigmoid(gates[:, :H])
            f = jax.nn.sigmoid(gates[:, H:2 * H])
            g = jnp.tanh(gates[:, 2 * H:3 * H])
            o = jax.nn.sigmoid(gates[:, 3 * H:])
            c = f * cs[l] + i * g
            hh = o * jnp.tanh(c)
            hs[l] = hh
            cs[l] = c
            inp_t = hh
        q = hs[2]
        e = out @ q.T
        emax = jnp.max(e, axis=0, keepdims=True)
        ex = jnp.exp(e - emax)
        alpha = ex / jnp.sum(ex, axis=0, keepdims=True)
        rvec = jnp.sum(alpha * out, axis=0, keepdims=True)
        q_star = jnp.concatenate([q, rvec], axis=1)
    y = jnp.maximum(q_star @ wr1[...] + br1[...], 0.0)
    y_ref[...] = y @ wr2[...] + br2[...]


_s2s = pl.pallas_call(
    _s2s_body,
    out_shape=jax.ShapeDtypeStruct((1, H), jnp.float32),
)


def kernel(x, edge_index, W0, b0, Wmsg, gru_Wi, gru_Wh, gru_bi, gru_bh,
           lstm_Wi0, lstm_Wh0, lstm_b0, lstm_Wi1, lstm_Wh1, lstm_b1,
           lstm_Wi2, lstm_Wh2, lstm_b2, Wr1, br1, Wr2, br2):
    src = edge_index[0]
    dst = edge_index[1]
    npad = _EPAD - E
    srcr = jnp.concatenate(
        [src, jnp.zeros((npad,), jnp.int32)]).reshape(_TOTCH, _CH)
    dstr = jnp.concatenate(
        [dst, jnp.full((npad,), N, jnp.int32)]).reshape(_TOTCH, _CH)

    out0, msg = _encode(x, W0, b0.reshape(1, H), Wmsg)
    h = out0
    bi = gru_bi.reshape(1, 3 * H)
    bh = gru_bh.reshape(1, 3 * H)
    for step in range(STEPS):
        partial = _get_sc_scatter()(msg, srcr, dstr)
        call = _gru_last if step == STEPS - 1 else _gru
        h, msg = call(partial, h, gru_Wi, gru_Wh, bi, bh, Wmsg)
    y = _s2s(h, lstm_Wi0, lstm_Wh0, lstm_b0.reshape(1, 4 * H),
             lstm_Wi1, lstm_Wh1, lstm_b1.reshape(1, 4 * H),
             lstm_Wi2, lstm_Wh2, lstm_b2.reshape(1, 4 * H),
             Wr1, br1.reshape(1, H), Wr2, br2.reshape(1, H))
    return y


# speedup vs baseline: 3.9851x; 2.8681x over previous
import functools

import jax
import jax.numpy as jnp
from jax import lax
from jax.experimental import pallas as pl
from jax.experimental.pallas import tpu as pltpu
from jax.experimental.pallas import tpu_sc as plsc

N = 10000
E = 320000
FT = 128
H = 64
STEPS = 6
S2S_STEPS = 6

_NC = 2
_NS = 16
_NW = _NC * _NS
_CH = 80
_RING = 5
_NCHUNK = 125
_TOTCH = _NW * _NCHUNK
_NACC = N + 16
_ZROWS = _NACC // _NS
_OROWS = N // _NS


def _sc_scatter_body(msg_hbm, src_hbm, dst_hbm, out_hbm,
                     srcv, dstv, rows, acc, *sems):
    c = lax.axis_index("c")
    s = lax.axis_index("s")
    gsems = sems[:_RING]
    ssems = sems[_RING:]

    start = (s * _NC + c) * _NCHUNK
    d1 = pltpu.async_copy(src_hbm.at[pl.ds(start, _NCHUNK)], srcv, gsems[0])
    d2 = pltpu.async_copy(dst_hbm.at[pl.ds(start, _NCHUNK)], dstv, gsems[1])

    def _zrow(i, carry):
        for k in range(4):
            rows[0, i, pl.ds(k * 16, 16)] = jnp.zeros((16,), jnp.float32)
        return carry
    lax.fori_loop(0, _CH, _zrow, 0)
    zb = s * _ZROWS
    nfull = _ZROWS // _CH
    for q in range(nfull):
        pltpu.sync_copy(rows.at[0], acc.at[pl.ds(zb + q * _CH, _CH)])
    rem = _ZROWS - nfull * _CH
    if rem:
        pltpu.sync_copy(rows.at[0, pl.ds(0, rem)],
                        acc.at[pl.ds(zb + nfull * _CH, rem)])
    d1.wait()
    d2.wait()
    plsc.subcore_barrier()

    def _wait_scatter(b):
        pltpu.make_async_copy(
            rows.at[b], acc.at[dstv.at[0]], ssems[b]).wait()

    def _group(g, carry):
        gds = []
        for b in range(_RING):
            j = g * _RING + b
            lax.cond(g > 0, lambda b=b: _wait_scatter(b), lambda: None)
            gds.append(pltpu.async_copy(
                msg_hbm.at[srcv.at[j]], rows.at[b], gsems[b]))
        for b in range(_RING):
            j = g * _RING + b
            gds[b].wait()
            pltpu.async_copy(rows.at[b], acc.at[dstv.at[j]], ssems[b],
                             add=True)
        return carry
    lax.fori_loop(0, _NCHUNK // _RING, _group, 0)
    for b in range(_RING):
        _wait_scatter(b)

    plsc.subcore_barrier()
    ob = s * _OROWS
    pltpu.sync_copy(acc.at[pl.ds(ob, _OROWS)], out_hbm.at[c, pl.ds(ob, _OROWS)])


@functools.cache
def _get_sc_scatter():
    return functools.partial(
        pl.kernel,
        out_type=jax.ShapeDtypeStruct((_NC, N, H), jnp.float32),
        mesh=plsc.VectorSubcoreMesh(core_axis_name="c", subcore_axis_name="s",
                                    num_cores=_NC, num_subcores=_NS),
        scratch_types=[
            pltpu.VMEM((_NCHUNK, _CH), jnp.int32),
            pltpu.VMEM((_NCHUNK, _CH), jnp.int32),
            pltpu.VMEM((_RING, _CH, H), jnp.float32),
            pltpu.VMEM_SHARED((_NACC, H), jnp.float32),
        ] + [pltpu.SemaphoreType.DMA] * (2 * _RING),
        compiler_params=pltpu.CompilerParams(use_tc_tiling_on_sc=False),
    )(_sc_scatter_body)


def _encode_body(x_ref, w0_ref, b0_ref, wmsg_ref, out_ref, msg_ref):
    o = jnp.maximum(x_ref[...] @ w0_ref[...] + b0_ref[...], 0.0)
    out_ref[...] = o
    msg_ref[...] = o @ wmsg_ref[...]


_encode = pl.pallas_call(
    _encode_body,
    out_shape=[jax.ShapeDtypeStruct((N, H), jnp.float32),
               jax.ShapeDtypeStruct((N, H), jnp.float32)],
)


def _gru_body(p_ref, h_ref, wi_ref, wh_ref, bi_ref, bh_ref, wmsg_ref,
              h_out_ref, msg_ref, *, last):
    m = jnp.maximum(p_ref[0] + p_ref[1], 0.0)
    h = h_ref[...]
    gx = m @ wi_ref[...] + bi_ref[...]
    gh = h @ wh_ref[...] + bh_ref[...]
    r = jax.nn.sigmoid(gx[:, :H] + gh[:, :H])
    z = jax.nn.sigmoid(gx[:, H:2 * H] + gh[:, H:2 * H])
    n = jnp.tanh(gx[:, 2 * H:] + r * gh[:, 2 * H:])
    hn = (1.0 - z) * n + z * h
    h_out_ref[...] = hn
    if last:
        msg_ref[...] = jnp.zeros_like(hn)
    else:
        msg_ref[...] = hn @ wmsg_ref[...]


_gru = pl.pallas_call(
    functools.partial(_gru_body, last=False),
    out_shape=[jax.ShapeDtypeStruct((N, H), jnp.float32),
               jax.ShapeDtypeStruct((N, H), jnp.float32)],
)

_gru_last = pl.pallas_call(
    functools.partial(_gru_body, last=True),
    out_shape=[jax.ShapeDtypeStruct((N, H), jnp.float32),
               jax.ShapeDtypeStruct((N, H), jnp.float32)],
)


def _s2s_body(out_ref, wi0, wh0, b0r, wi1, wh1, b1r, wi2, wh2, b2r,
              wr1, br1, wr2, br2, y_ref):
    out = out_ref[...]
    wi = (wi0[...], wi1[...], wi2[...])
    wh = (wh0[...], wh1[...], wh2[...])
    bb = (b0r[...], b1r[...], b2r[...])
    hs = [jnp.zeros((1, H), jnp.float32) for _ in range(3)]
    cs = [jnp.zeros((1, H), jnp.float32) for _ in range(3)]
    q_star = jnp.zeros((1, 2 * H), jnp.float32)
    for _ in range(S2S_STEPS):
        inp_t = q_star
        for l in range(3):
            gates = inp_t @ wi[l] + hs[l] @ wh[l] + bb[l]
            i = jax.nn.sigmoid(gates[:, :H])
            f = jax.nn.sigmoid(gates[:, H:2 * H])
            g = jnp.tanh(gates[:, 2 * H:3 * H])
            o = jax.nn.sigmoid(gates[:, 3 * H:])
            c = f * cs[l] + i * g
            hh = o * jnp.tanh(c)
            hs[l] = hh
            cs[l] = c
            inp_t = hh
        q = hs[2]
        e = out @ q.T
        emax = jnp.max(e, axis=0, keepdims=True)
        ex = jnp.exp(e - emax)
        alpha = ex / jnp.sum(ex, axis=0, keepdims=True)
        rvec = jnp.sum(alpha * out, axis=0, keepdims=True)
        q_star = jnp.concatenate([q, rvec], axis=1)
    y = jnp.maximum(q_star @ wr1[...] + br1[...], 0.0)
    y_ref[...] = y @ wr2[...] + br2[...]


_s2s = pl.pallas_call(
    _s2s_body,
    out_shape=jax.ShapeDtypeStruct((1, H), jnp.float32),
)


def kernel(x, edge_index, W0, b0, Wmsg, gru_Wi, gru_Wh, gru_bi, gru_bh,
           lstm_Wi0, lstm_Wh0, lstm_b0, lstm_Wi1, lstm_Wh1, lstm_b1,
           lstm_Wi2, lstm_Wh2, lstm_b2, Wr1, br1, Wr2, br2):
    src = edge_index[0]
    dst = edge_index[1]
    srcr = src.reshape(_TOTCH, _CH)
    dstr = dst.reshape(_TOTCH, _CH)

    out0, msg = _encode(x, W0, b0.reshape(1, H), Wmsg)
    h = out0
    bi = gru_bi.reshape(1, 3 * H)
    bh = gru_bh.reshape(1, 3 * H)
    for step in range(STEPS):
        partial = _get_sc_scatter()(msg, srcr, dstr)
        call = _gru_last if step == STEPS - 1 else _gru
        h, msg = call(partial, h, gru_Wi, gru_Wh, bi, bh, Wmsg)
    y = _s2s(h, lstm_Wi0, lstm_Wh0, lstm_b0.reshape(1, 4 * H),
             lstm_Wi1, lstm_Wh1, lstm_b1.reshape(1, 4 * H),
             lstm_Wi2, lstm_Wh2, lstm_b2.reshape(1, 4 * H),
             Wr1, br1.reshape(1, H), Wr2, br2.reshape(1, H))
    return y
